# baseline (speedup 1.0000x reference)
import jax
import jax.numpy as jnp
from jax import lax
from jax.experimental import pallas as pl
from jax.experimental.pallas import tpu as pltpu
from jax.experimental.pallas import tpu_sc as plsc

N_TABLES = 4
N_LEVELS = 16
F_PER_LEVEL = 2
LOG2_HASH = 19
HASH_SIZE = 2 ** LOG2_HASH
MASK = HASH_SIZE - 1
BASE_RES = 16
PER_LEVEL_SCALE = 1.4472692012786865
N_POINTS = 131072
PRIME1 = 2654435761
PRIME2 = 805459861

NC = 2
NS = 16
NW = NC * NS
PPW = N_POINTS // NW
C = 16
NCHUNK = PPW // C
ROWS = N_LEVELS * 8 * C
D_OUT = N_LEVELS * F_PER_LEVEL
NGRP = NCHUNK // 8

import numpy as _np
RES = [int(_np.floor(BASE_RES * (PER_LEVEL_SCALE ** l))) for l in range(N_LEVELS)]

_P1 = PRIME1 - (1 << 32)
_P2 = PRIME2

_DNUMS = lax.GatherDimensionNumbers(
    offset_dims=(), collapsed_slice_dims=(0,), start_index_map=(0,))


def _vperm(v, idx16):
    return lax.gather(v, idx16[:, None], _DNUMS, (1,),
                      mode=lax.GatherScatterMode.PROMISE_IN_BOUNDS)


def _body(in_h, code_h, tab_h, out_h,
          xs_v, ys_v, zs_v, c0_v, c1_v, c2_v, c3_v,
          idx_v, bw2_v, rows_v, crep_v, out_v, semA, semB):
    wid = lax.axis_index("s") * NC + lax.axis_index("c")
    base0 = wid * PPW
    pltpu.sync_copy(in_h.at[0, 0, pl.ds(base0, PPW)], xs_v)
    pltpu.sync_copy(in_h.at[1, 0, pl.ds(base0, PPW)], ys_v)
    pltpu.sync_copy(in_h.at[2, 0, pl.ds(base0, PPW)], zs_v)
    pltpu.sync_copy(code_h.at[0, 0, pl.ds(base0, PPW)], c0_v)
    pltpu.sync_copy(code_h.at[1, 0, pl.ds(base0, PPW)], c1_v)
    pltpu.sync_copy(code_h.at[2, 0, pl.ds(base0, PPW)], c2_v)
    pltpu.sync_copy(code_h.at[3, 0, pl.ds(base0, PPW)], c3_v)

    iota16 = lax.iota(jnp.int32, 16)
    splat = [jnp.full((16,), p, dtype=jnp.int32) for p in range(C)]
    sh2 = (iota16 + 2) & 15
    sh4 = (iota16 + 4) & 15
    sh8 = (iota16 + 8) & 15
    hrows = lax.shift_right_logical(iota16, 3) * ROWS
    posbase = (iota16 & 1) * 128
    m2 = iota16 < 2
    crep_pos = iota16 * 16

    def _issue(g, s, sem):
        cb = g * C
        x = xs_v[pl.ds(cb, C)]
        y = ys_v[pl.ds(cb, C)]
        z = zs_v[pl.ds(cb, C)]
        for l in range(N_LEVELS):
            res = jnp.float32(RES[l])
            px = x * res
            py = y * res
            pz = z * res
            ix = px.astype(jnp.int32)
            iy = py.astype(jnp.int32)
            iz = pz.astype(jnp.int32)
            wx = px - ix.astype(jnp.float32)
            wy = py - iy.astype(jnp.float32)
            wz = pz - iz.astype(jnp.float32)
            ox = jnp.float32(1.0) - wx
            oy = jnp.float32(1.0) - wy
            oz = jnp.float32(1.0) - wz
            hy0 = iy * _P1
            hz0 = iz * _P2
            hy1 = (iy + 1) * _P1
            hz1 = (iz + 1) * _P2
            lbase = l * (HASH_SIZE // 2)
            for c in range(8):
                bx, by, bz = c & 1, (c >> 1) & 1, (c >> 2) & 1
                cx = ix + 1 if bx else ix
                h = cx ^ (hy1 if by else hy0) ^ (hz1 if bz else hz0)
                gidx = h & MASK
                prow = lax.shift_right_logical(gidx, 1) + lbase
                hf = (gidx & 1).astype(jnp.float32)
                bw = (wx if bx else ox) * (wy if by else oy) * (wz if bz else oz)
                off = (l * 8 + c) * C
                idx_v[s, pl.ds(off, C)] = prow
                bw2_v[s, pl.ds(off, C)] = bw * (jnp.float32(1.0) - hf)
                bw2_v[s, pl.ds(ROWS + off, C)] = bw * hf
        q = [c0_v[pl.ds(cb, C)], c1_v[pl.ds(cb, C)],
             c2_v[pl.ds(cb, C)], c3_v[pl.ds(cb, C)]]
        for t in range(4):
            for u in range(4):
                plsc.store_scatter(crep_v.at[s],
                                   [crep_pos + (4 * t + u)], q[t])
        for k in range(ROWS // 128):
            pltpu.async_copy(
                tab_h.at[idx_v.at[s, pl.ds(k * 128, 128)]],
                rows_v.at[s, pl.ds(k * 128, 128), :], sem)

    def _consume(g, s, sem):
        pltpu.make_async_copy(tab_h.at[pl.ds(0, ROWS), :],
                              rows_v.at[s], sem).wait()
        col = (g & 7) * C

        @pl.loop(0, N_LEVELS)
        def _lvl(l):
            acc = [jnp.zeros((16,), jnp.float32)] * C
            for c in range(8):
                rbase = (l * 8 + c) * C
                bwlo = bw2_v[s, pl.ds(rbase, C)]
                bwhi = bw2_v[s, pl.ds(ROWS + rbase, C)]
                for p in range(C):
                    lo = _vperm(bwlo, splat[p])
                    hi = _vperm(bwhi, splat[p])
                    bwsel = jnp.where(m2crep, lo, hi)
                    row = rows_v[s, rbase + p, :]
                    acc[p] = acc[p] + row * bwsel
            for p in range(C):
                cp = crep_v[s, pl.ds(p * 16, 16)]
                m = acc[p] * cp
                s1 = m + _vperm(m, sh2)
                s2 = s1 + _vperm(s1, sh4)
                s3 = s2 + _vperm(s2, sh8)
                pos = posbase + (2 * l * 128 + col + p)
                plsc.store_scatter(out_v, [pos], s3, mask=m2)

        @pl.when((g & 7) == 7)
        def _flush():
            grp = lax.shift_right_logical(g, 3)
            pltpu.sync_copy(out_v,
                            out_h.at[pl.ds((wid * NGRP + grp) * (D_OUT * 128),
                                           D_OUT * 128)])

    m2crep = (lax.shift_right_logical(iota16, 1) & 1) < 1

    _issue(0, 0, semA)

    @pl.loop(0, NCHUNK // 2 - 1)
    def _pair(mi):
        a = 2 * mi
        _issue(a + 1, 1, semB)
        _consume(a, 0, semA)
        _issue(a + 2, 0, semA)
        _consume(a + 1, 1, semB)

    _issue(NCHUNK - 1, 1, semB)
    _consume(NCHUNK - 2, 0, semA)
    _consume(NCHUNK - 1, 1, semB)


def _prep_body(tab4_h, out_h, stage, stage2):
    sid = lax.axis_index("s")
    wid = sid * NC + lax.axis_index("c")
    l = lax.shift_right_logical(wid, 1)
    ibase = (wid & 1) * (HASH_SIZE // 4)
    B = 512

    @pl.loop(0, (HASH_SIZE // 4) // B)
    def _blk(m):
        i0 = ibase + m * B
        for t in range(N_TABLES):
            pltpu.sync_copy(tab4_h.at[t, l, pl.ds(i0, B), :], stage.at[t])
        for t in range(N_TABLES):
            pltpu.sync_copy(stage.at[t], stage2.at[sid, :, pl.ds(4 * t, 4)])
        r0 = l * (HASH_SIZE // 2) + i0
        pltpu.sync_copy(stage2.at[sid], out_h.at[pl.ds(r0, B), :])


@jax.jit
def _prep(tab4):
    mesh = plsc.VectorSubcoreMesh(core_axis_name="c", subcore_axis_name="s")
    f = pl.kernel(
        _prep_body,
        out_type=jax.ShapeDtypeStruct(
            (N_LEVELS * HASH_SIZE // 2, 2 * N_TABLES * F_PER_LEVEL),
            jnp.float32),
        mesh=mesh,
        compiler_params=pltpu.CompilerParams(needs_layout_passes=False,
                                             use_tc_tiling_on_sc=False),
        scratch_types=[
            pltpu.VMEM((N_TABLES, 512, 4), jnp.float32),
            pltpu.VMEM_SHARED((NS, 512, 16), jnp.float32),
        ],
    )
    return f(tab4)


@jax.jit
def _run(in_tensor, code, tab):
    mesh = plsc.VectorSubcoreMesh(core_axis_name="c", subcore_axis_name="s")
    f = pl.kernel(
        _body,
        out_type=jax.ShapeDtypeStruct((N_POINTS * D_OUT,), jnp.float32),
        mesh=mesh,
        compiler_params=pltpu.CompilerParams(needs_layout_passes=False,
                                             use_tc_tiling_on_sc=False),
        scratch_types=[
            pltpu.VMEM((PPW,), jnp.float32),
            pltpu.VMEM((PPW,), jnp.float32),
            pltpu.VMEM((PPW,), jnp.float32),
            pltpu.VMEM((PPW,), jnp.float32),
            pltpu.VMEM((PPW,), jnp.float32),
            pltpu.VMEM((PPW,), jnp.float32),
            pltpu.VMEM((PPW,), jnp.float32),
            pltpu.VMEM((2, ROWS), jnp.int32),
            pltpu.VMEM((2, 2 * ROWS), jnp.float32),
            pltpu.VMEM((2, ROWS, 16), jnp.float32),
            pltpu.VMEM((2, 256), jnp.float32),
            pltpu.VMEM((D_OUT * 128,), jnp.float32),
            pltpu.SemaphoreType.DMA,
            pltpu.SemaphoreType.DMA,
        ],
    )
    return f(in_tensor, code, tab)


def kernel(in_tensor, conditioning_code, tables):
    tab4 = tables.reshape(N_TABLES, N_LEVELS, HASH_SIZE // 2, 2 * F_PER_LEVEL)
    tab = _prep(tab4)
    inT = in_tensor.T.reshape(3, 1, N_POINTS)
    codeT = conditioning_code.T.reshape(N_TABLES, 1, N_POINTS)
    out = _run(inT, codeT, tab)
    out = out.reshape(NW, NGRP, D_OUT, 128).transpose(0, 1, 3, 2)
    return out.reshape(N_POINTS, D_OUT)


# speedup vs baseline: 8.8516x; 8.8516x over previous
import jax
import jax.numpy as jnp
from jax import lax
from jax.experimental import pallas as pl
from jax.experimental.pallas import tpu as pltpu
from jax.experimental.pallas import tpu_sc as plsc

N_TABLES = 4
N_LEVELS = 16
F_PER_LEVEL = 2
LOG2_HASH = 19
HASH_SIZE = 2 ** LOG2_HASH
MASK = HASH_SIZE - 1
BASE_RES = 16
PER_LEVEL_SCALE = 1.4472692012786865
N_POINTS = 131072
PRIME1 = 2654435761
PRIME2 = 805459861

NC = 2
NS = 16
NW = NC * NS
PPW = N_POINTS // NW
C = 16
NCHUNK = PPW // C
ROWS = N_LEVELS * 8 * C
D_OUT = N_LEVELS * F_PER_LEVEL
NGRP = NCHUNK // 8

import numpy as _np
RES = [int(_np.floor(BASE_RES * (PER_LEVEL_SCALE ** l))) for l in range(N_LEVELS)]

_P1 = PRIME1 - (1 << 32)
_P2 = PRIME2

_DNUMS = lax.GatherDimensionNumbers(
    offset_dims=(), collapsed_slice_dims=(0,), start_index_map=(0,))


def _vperm(v, idx16):
    return lax.gather(v, idx16[:, None], _DNUMS, (1,),
                      mode=lax.GatherScatterMode.PROMISE_IN_BOUNDS)


def _body(in_h, code_h, tab_h, out_h,
          xs_v, ys_v, zs_v, c0_v, c1_v, c2_v, c3_v,
          idx_v, bw2_v, rows_v, crep_v, out_v, semA, semB):
    wid = lax.axis_index("s") * NC + lax.axis_index("c")
    base0 = wid * PPW
    pltpu.sync_copy(in_h.at[0, 0, pl.ds(base0, PPW)], xs_v)
    pltpu.sync_copy(in_h.at[1, 0, pl.ds(base0, PPW)], ys_v)
    pltpu.sync_copy(in_h.at[2, 0, pl.ds(base0, PPW)], zs_v)
    pltpu.sync_copy(code_h.at[0, 0, pl.ds(base0, PPW)], c0_v)
    pltpu.sync_copy(code_h.at[1, 0, pl.ds(base0, PPW)], c1_v)
    pltpu.sync_copy(code_h.at[2, 0, pl.ds(base0, PPW)], c2_v)
    pltpu.sync_copy(code_h.at[3, 0, pl.ds(base0, PPW)], c3_v)

    iota16 = lax.iota(jnp.int32, 16)
    splat = [jnp.full((16,), p, dtype=jnp.int32) for p in range(C)]
    sh2 = (iota16 + 2) & 15
    sh4 = (iota16 + 4) & 15
    sh8 = (iota16 + 8) & 15
    hrows = lax.shift_right_logical(iota16, 3) * ROWS
    posbase = (iota16 & 1) * 128
    m2 = iota16 < 2
    crep_pos = iota16 * 16

    def _issue(g, s, sem):
        cb = g * C
        x = xs_v[pl.ds(cb, C)]
        y = ys_v[pl.ds(cb, C)]
        z = zs_v[pl.ds(cb, C)]
        for l in range(N_LEVELS):
            res = jnp.float32(RES[l])
            px = x * res
            py = y * res
            pz = z * res
            ix = px.astype(jnp.int32)
            iy = py.astype(jnp.int32)
            iz = pz.astype(jnp.int32)
            wx = px - ix.astype(jnp.float32)
            wy = py - iy.astype(jnp.float32)
            wz = pz - iz.astype(jnp.float32)
            ox = jnp.float32(1.0) - wx
            oy = jnp.float32(1.0) - wy
            oz = jnp.float32(1.0) - wz
            hy0 = iy * _P1
            hz0 = iz * _P2
            hy1 = (iy + 1) * _P1
            hz1 = (iz + 1) * _P2
            lbase = l * (HASH_SIZE // 2)
            for c in range(8):
                bx, by, bz = c & 1, (c >> 1) & 1, (c >> 2) & 1
                cx = ix + 1 if bx else ix
                h = cx ^ (hy1 if by else hy0) ^ (hz1 if bz else hz0)
                gidx = h & MASK
                prow = lax.shift_right_logical(gidx, 1) + lbase
                hf = (gidx & 1).astype(jnp.float32)
                bw = (wx if bx else ox) * (wy if by else oy) * (wz if bz else oz)
                off = (l * 8 + c) * C
                idx_v[s, pl.ds(off, C)] = prow
                bw2_v[s, pl.ds(off, C)] = bw * (jnp.float32(1.0) - hf)
                bw2_v[s, pl.ds(ROWS + off, C)] = bw * hf
        q = [c0_v[pl.ds(cb, C)], c1_v[pl.ds(cb, C)],
             c2_v[pl.ds(cb, C)], c3_v[pl.ds(cb, C)]]
        for t in range(4):
            for u in range(4):
                plsc.store_scatter(crep_v.at[s],
                                   [crep_pos + (4 * t + u)], q[t])
        for k in range(ROWS // 128):
            pltpu.async_copy(
                tab_h.at[idx_v.at[s, pl.ds(k * 128, 128)]],
                rows_v.at[s, pl.ds(k * 128, 128), :], sem)

    def _consume(g, s, sem):
        pltpu.make_async_copy(tab_h.at[pl.ds(0, ROWS), :],
                              rows_v.at[s], sem).wait()
        col = (g & 7) * C

        @pl.loop(0, N_LEVELS)
        def _lvl(l):
            acc = [jnp.zeros((16,), jnp.float32)] * C
            for c in range(8):
                rbase = (l * 8 + c) * C
                bwlo = bw2_v[s, pl.ds(rbase, C)]
                bwhi = bw2_v[s, pl.ds(ROWS + rbase, C)]
                for p in range(C):
                    lo = _vperm(bwlo, splat[p])
                    hi = _vperm(bwhi, splat[p])
                    bwsel = jnp.where(m2crep, lo, hi)
                    row = rows_v[s, rbase + p, :]
                    acc[p] = acc[p] + row * bwsel
            for p in range(C):
                cp = crep_v[s, pl.ds(p * 16, 16)]
                m = acc[p] * cp
                s1 = m + _vperm(m, sh2)
                s2 = s1 + _vperm(s1, sh4)
                s3 = s2 + _vperm(s2, sh8)
                pos = posbase + (2 * l * 128 + col + p)
                plsc.store_scatter(out_v, [pos], s3, mask=m2)

        @pl.when((g & 7) == 7)
        def _flush():
            grp = lax.shift_right_logical(g, 3)
            pltpu.sync_copy(out_v,
                            out_h.at[pl.ds((wid * NGRP + grp) * (D_OUT * 128),
                                           D_OUT * 128)])

    m2crep = (lax.shift_right_logical(iota16, 1) & 1) < 1

    _issue(0, 0, semA)

    @pl.loop(0, NCHUNK // 2 - 1)
    def _pair(mi):
        a = 2 * mi
        _issue(a + 1, 1, semB)
        _consume(a, 0, semA)
        _issue(a + 2, 0, semA)
        _consume(a + 1, 1, semB)

    _issue(NCHUNK - 1, 1, semB)
    _consume(NCHUNK - 2, 0, semA)
    _consume(NCHUNK - 1, 1, semB)


@jax.jit
def _run(in_tensor, code, tab):
    mesh = plsc.VectorSubcoreMesh(core_axis_name="c", subcore_axis_name="s")
    f = pl.kernel(
        _body,
        out_type=jax.ShapeDtypeStruct((N_POINTS * D_OUT,), jnp.float32),
        mesh=mesh,
        compiler_params=pltpu.CompilerParams(needs_layout_passes=False,
                                             use_tc_tiling_on_sc=False),
        scratch_types=[
            pltpu.VMEM((PPW,), jnp.float32),
            pltpu.VMEM((PPW,), jnp.float32),
            pltpu.VMEM((PPW,), jnp.float32),
            pltpu.VMEM((PPW,), jnp.float32),
            pltpu.VMEM((PPW,), jnp.float32),
            pltpu.VMEM((PPW,), jnp.float32),
            pltpu.VMEM((PPW,), jnp.float32),
            pltpu.VMEM((2, ROWS), jnp.int32),
            pltpu.VMEM((2, 2 * ROWS), jnp.float32),
            pltpu.VMEM((2, ROWS, 16), jnp.float32),
            pltpu.VMEM((2, 256), jnp.float32),
            pltpu.VMEM((D_OUT * 128,), jnp.float32),
            pltpu.SemaphoreType.DMA,
            pltpu.SemaphoreType.DMA,
        ],
    )
    return f(in_tensor, code, tab)


def kernel(in_tensor, conditioning_code, tables):
    tab = jnp.transpose(
        tables.reshape(N_TABLES, N_LEVELS, HASH_SIZE // 2, 2 * F_PER_LEVEL),
        (1, 2, 0, 3)).reshape(N_LEVELS * HASH_SIZE // 2,
                              2 * N_TABLES * F_PER_LEVEL)
    inT = in_tensor.T.reshape(3, 1, N_POINTS)
    codeT = conditioning_code.T.reshape(N_TABLES, 1, N_POINTS)
    out = _run(inT, codeT, tab)
    out = out.reshape(NW, NGRP, D_OUT, 128).transpose(0, 1, 3, 2)
    return out.reshape(N_POINTS, D_OUT)
